# baseline (speedup 1.0000x reference)
import functools

import jax
import jax.numpy as jnp
from jax import lax
from jax.experimental import pallas as pl
from jax.experimental.pallas import tpu as pltpu
from jax.experimental.pallas import tpu_sc as plsc

_B, _F, _D, _V = 16384, 129, 128, 100
_NC, _NS, _L = 2, 16, 16
_NW = _NC * _NS
_RPW = _B // _NW
_BLK = 128
_NBLK = _RPW // _BLK


def _body(x_hbm, table_hbm, out_hbm, xbuf, idx_v, emb, sem):
    wid = lax.axis_index("s") * _NC + lax.axis_index("c")
    lanes = lax.iota(jnp.int32, _L)
    for k in range(_NBLK):
        base = wid * _RPW + k * _BLK
        pltpu.sync_copy(x_hbm.at[pl.ds(base, _BLK), :], xbuf)
        for i in range(_BLK // _L):
            acc = jnp.zeros((_L,), jnp.float32)
            for j in range(_L):
                s = xbuf[i * _L + j, pl.ds(_F - _L, _L)][_L - 1]
                acc = jnp.where(lanes == j, s, acc)
            idx_v[pl.ds(i * _L, _L)] = acc.astype(jnp.int32)
        pltpu.async_copy(table_hbm.at[idx_v], emb, sem).wait()
        pltpu.sync_copy(xbuf.at[:, pl.ds(0, _D)],
                        out_hbm.at[pl.ds(base, _BLK), pl.ds(0, _D)])
        pltpu.sync_copy(emb, out_hbm.at[pl.ds(base, _BLK), pl.ds(_D, _D)])


@jax.jit
def kernel(x, table):
    mesh = plsc.VectorSubcoreMesh(core_axis_name="c", subcore_axis_name="s")
    f = pl.kernel(
        _body,
        mesh=mesh,
        out_type=jax.ShapeDtypeStruct((_B, 2 * _D), jnp.float32),
        scratch_types=[
            pltpu.VMEM((_BLK, _F), jnp.float32),
            pltpu.VMEM((_BLK,), jnp.int32),
            pltpu.VMEM((_BLK, _D), jnp.float32),
            pltpu.SemaphoreType.DMA,
        ],
        compiler_params=pltpu.CompilerParams(use_tc_tiling_on_sc=False),
    )
    return f(x, table)
